# baseline (speedup 1.0000x reference)
import functools

import jax
import jax.numpy as jnp
from jax import lax
from jax.experimental import pallas as pl
from jax.experimental.pallas import tpu as pltpu
from jax.experimental.pallas import tpu_sc as plsc

_B, _L, _D = 16, 4096, 1024
_LANES = 16
_NT = 16
_CH = 40
_SPLIT = 10
_CHT = 384
_K = 6

_mesh = plsc.VectorSubcoreMesh(core_axis_name="c", subcore_axis_name="s")


@functools.partial(
    pl.kernel,
    mesh=_mesh,
    compiler_params=pltpu.CompilerParams(needs_layout_passes=False),
    out_type=jax.ShapeDtypeStruct((_B, _D), jnp.float32),
    scratch_types=[
        pltpu.VMEM((_LANES,), jnp.int32),
        pltpu.VMEM((_CH, _D), jnp.float32),
        pltpu.VMEM((_CH, _D), jnp.float32),
        pltpu.VMEM((1, 1, _D), jnp.float32),
        pltpu.VMEM((_NT, 1, _D), jnp.float32),
        pltpu.VMEM((_D,), jnp.float32),
        pltpu.VMEM_SHARED((_NT, _B, _D), jnp.float32),
        pltpu.SemaphoreType.DMA,
        pltpu.SemaphoreType.DMA,
    ],
)
def _pool(feat, lens_hbm, out, len_v, buf0, buf1, acc, tmp, orow, shparts,
          sem0, sem1):
    c = lax.axis_index("c")
    s = lax.axis_index("s")

    pltpu.sync_copy(lens_hbm, len_v)
    lens = jnp.where(lax.iota(jnp.int32, _LANES) >= _SPLIT, len_v[...], 0)
    cum = jnp.cumsum(lens)
    total = jnp.sum(lens)
    half = total // 2
    rows0 = jnp.max(jnp.where(cum <= half, cum, 0))
    k = jnp.sum((cum <= half).astype(jnp.int32))

    r_base = c * rows0
    r_sc = jnp.where(c == 0, rows0, total - rows0)
    r0 = r_base + (s * r_sc) // _NT
    r1 = r_base + ((s + 1) * r_sc) // _NT

    zero = jnp.zeros((_LANES,), jnp.float32)

    def zero_acc():
        for d in range(_D // _LANES):
            acc[0, 0, pl.ds(d * _LANES, _LANES)] = zero

    zero_acc()

    def zslot(b, carry):
        pltpu.sync_copy(acc, shparts.at[pl.ds(s, 1), pl.ds(b, 1)])
        return carry

    lax.fori_loop(0, _B, zslot, 0)

    def piece_body(r):
        start_i = jnp.max(jnp.where(cum <= r, cum, 0))
        end_i = jnp.min(jnp.where(cum > r, cum, total))
        i = jnp.sum((cum <= r).astype(jnp.int32))
        a = r - start_i
        pe = jnp.minimum(end_i, r1)
        n = pe - r

        zero_acc()

        a0 = (a // 8) * 8
        aend = a + n
        nch = (aend - a0 + _CH - 1) // _CH

        def st_of(j):
            return jnp.minimum(a0 + j * _CH, _L - _CH)

        def start_dma(j, buf, sem):
            pltpu.make_async_copy(
                feat.at[i, pl.ds(st_of(j), _CH), :], buf, sem
            ).start()

        def wait_dma(buf, sem):
            pltpu.make_async_copy(
                feat.at[i, pl.ds(0, _CH), :], buf, sem
            ).wait()

        def accum(buf, j):
            st = st_of(j)
            lo = jnp.maximum(a, a0 + j * _CH) - st
            hi = jnp.minimum(aend, st + _CH) - st
            for dc in range(_D // 128):
                base = dc * 128

                def body(t, a8, base=base):
                    return tuple(
                        a8[q] + buf[t, pl.ds(base + q * _LANES, _LANES)]
                        for q in range(8)
                    )

                a8 = plsc.parallel_loop(lo, hi, 1, unroll=2, carry=(zero,) * 8)(body)
                for q in range(8):
                    plsc.addupdate(
                        acc.at[0, 0, pl.ds(base + q * _LANES, _LANES)], a8[q]
                    )

        start_dma(0, buf0, sem0)

        def pair_body(p, carry):
            j0 = 2 * p
            j1 = j0 + 1

            @pl.when(j1 < nch)
            def _():
                start_dma(j1, buf1, sem1)

            wait_dma(buf0, sem0)
            accum(buf0, j0)

            @pl.when(j1 < nch)
            def _():
                @pl.when(j1 + 1 < nch)
                def _():
                    start_dma(j1 + 1, buf0, sem0)

                wait_dma(buf1, sem1)
                accum(buf1, j1)

            return carry

        lax.fori_loop(0, (nch + 1) // 2, pair_body, 0)

        pltpu.sync_copy(acc, shparts.at[pl.ds(s, 1), pl.ds(i, 1)])
        return pe

    lax.while_loop(lambda r: r < r1, piece_body, r0)

    plsc.subcore_barrier()

    mine = jnp.where(c == 0, (s >= _SPLIT) & (s < k), s >= k)

    @pl.when(mine)
    def _():
        pltpu.sync_copy(shparts.at[pl.ds(0, _NT), pl.ds(s, 1)], tmp)
        lsplat = plsc.load_gather(len_v, [jnp.full((_LANES,), s, jnp.int32)])
        recip = jnp.full((_LANES,), 1.0, jnp.float32) / lsplat.astype(jnp.float32)
        for dc in range(_D // 128):
            base = dc * 128

            def body(t, a8, base=base):
                return tuple(
                    a8[q] + tmp[t, 0, pl.ds(base + q * _LANES, _LANES)]
                    for q in range(8)
                )

            a8 = plsc.parallel_loop(0, _NT, 1, unroll=2, carry=(zero,) * 8)(body)
            for q in range(8):
                orow[pl.ds(base + q * _LANES, _LANES)] = a8[q] * recip
        pltpu.sync_copy(orow, out.at[s])


def _tc_pool(lens_smem, feat_hbm, out_ref, bufs, sems):
    def nch(b):
        return (lens_smem[b] + _CHT - 1) // _CHT

    n_total = lax.fori_loop(0, _SPLIT, lambda b, t: t + nch(b), 0)

    def adv(b, j):
        last = j + 1 >= nch(b)
        return jnp.where(last, b + 1, b), jnp.where(last, 0, j + 1)

    def issue(b, j, q):
        pltpu.make_async_copy(
            feat_hbm.at[b, pl.ds(j * _CHT, _CHT), :], bufs.at[q], sems.at[q]
        ).start()

    def prologue(q, st):
        b, j = st

        @pl.when(q < n_total)
        def _():
            issue(b, j, q)

        return adv(b, j)

    bi0, ji0 = lax.fori_loop(0, _K, prologue, (0, 0))

    def body(t, st):
        bc, jc, bi, ji, acc = st
        q = t % _K
        pltpu.make_async_copy(
            feat_hbm.at[0, pl.ds(0, _CHT), :], bufs.at[q], sems.at[q]
        ).wait()
        blen = lens_smem[bc]
        m = (
            lax.broadcasted_iota(jnp.int32, (8, _CHT), 1) < (blen - jc * _CHT)
        ).astype(jnp.float32)
        acc = acc + jnp.dot(m, bufs[q], preferred_element_type=jnp.float32)
        last = jc + 1 >= nch(bc)

        @pl.when(last)
        def _():
            out_ref[pl.ds(bc, 1), :] = acc[0:1, :] / blen.astype(jnp.float32)

        acc = jnp.where(last, jnp.zeros_like(acc), acc)

        @pl.when(t + _K < n_total)
        def _():
            issue(bi, ji, q)

        bi, ji = adv(bi, ji)
        bc, jc = adv(bc, jc)
        return bc, jc, bi, ji, acc

    lax.fori_loop(
        0,
        n_total,
        body,
        (0, 0, bi0, ji0, jnp.zeros((8, _D), jnp.float32)),
    )


_tc_call = pl.pallas_call(
    _tc_pool,
    in_specs=[
        pl.BlockSpec(memory_space=pltpu.SMEM),
        pl.BlockSpec(memory_space=pl.ANY),
    ],
    out_specs=pl.BlockSpec(memory_space=pltpu.VMEM),
    out_shape=jax.ShapeDtypeStruct((_SPLIT, _D), jnp.float32),
    scratch_shapes=[
        pltpu.VMEM((_K, _CHT, _D), jnp.float32),
        pltpu.SemaphoreType.DMA((_K,)),
    ],
)


def kernel(features, lengths):
    tc_out = _tc_call(lengths, features)
    sc_out = _pool(features, lengths)
    return lax.dynamic_update_slice(sc_out, tc_out, (0, 0))


# speedup vs baseline: 1.0353x; 1.0353x over previous
import functools

import jax
import jax.numpy as jnp
from jax import lax
from jax.experimental import pallas as pl
from jax.experimental.pallas import tpu as pltpu
from jax.experimental.pallas import tpu_sc as plsc

_B, _L, _D = 16, 4096, 1024
_LANES = 16
_NT = 16
_CH = 40
_SPLIT = 10
_CHT = 256
_K = 8

_mesh = plsc.VectorSubcoreMesh(core_axis_name="c", subcore_axis_name="s")


@functools.partial(
    pl.kernel,
    mesh=_mesh,
    compiler_params=pltpu.CompilerParams(needs_layout_passes=False),
    out_type=jax.ShapeDtypeStruct((_B, _D), jnp.float32),
    scratch_types=[
        pltpu.VMEM((_LANES,), jnp.int32),
        pltpu.VMEM((_CH, _D), jnp.float32),
        pltpu.VMEM((_CH, _D), jnp.float32),
        pltpu.VMEM((1, 1, _D), jnp.float32),
        pltpu.VMEM((_NT, 1, _D), jnp.float32),
        pltpu.VMEM((_D,), jnp.float32),
        pltpu.VMEM_SHARED((_NT, _B, _D), jnp.float32),
        pltpu.SemaphoreType.DMA,
        pltpu.SemaphoreType.DMA,
    ],
)
def _pool(feat, lens_hbm, out, len_v, buf0, buf1, acc, tmp, orow, shparts,
          sem0, sem1):
    c = lax.axis_index("c")
    s = lax.axis_index("s")

    pltpu.sync_copy(lens_hbm, len_v)
    lens = jnp.where(lax.iota(jnp.int32, _LANES) >= _SPLIT, len_v[...], 0)
    cum = jnp.cumsum(lens)
    total = jnp.sum(lens)
    half = total // 2
    rows0 = jnp.max(jnp.where(cum <= half, cum, 0))
    k = jnp.sum((cum <= half).astype(jnp.int32))

    r_base = c * rows0
    r_sc = jnp.where(c == 0, rows0, total - rows0)
    r0 = r_base + (s * r_sc) // _NT
    r1 = r_base + ((s + 1) * r_sc) // _NT

    zero = jnp.zeros((_LANES,), jnp.float32)

    def zero_acc():
        for d in range(_D // _LANES):
            acc[0, 0, pl.ds(d * _LANES, _LANES)] = zero

    zero_acc()

    def zslot(b, carry):
        pltpu.sync_copy(acc, shparts.at[pl.ds(s, 1), pl.ds(b, 1)])
        return carry

    lax.fori_loop(0, _B, zslot, 0)

    def piece_body(r):
        start_i = jnp.max(jnp.where(cum <= r, cum, 0))
        end_i = jnp.min(jnp.where(cum > r, cum, total))
        i = jnp.sum((cum <= r).astype(jnp.int32))
        a = r - start_i
        pe = jnp.minimum(end_i, r1)
        n = pe - r

        zero_acc()

        a0 = (a // 8) * 8
        aend = a + n
        nch = (aend - a0 + _CH - 1) // _CH

        def st_of(j):
            return jnp.minimum(a0 + j * _CH, _L - _CH)

        def start_dma(j, buf, sem):
            pltpu.make_async_copy(
                feat.at[i, pl.ds(st_of(j), _CH), :], buf, sem
            ).start()

        def wait_dma(buf, sem):
            pltpu.make_async_copy(
                feat.at[i, pl.ds(0, _CH), :], buf, sem
            ).wait()

        def accum(buf, j):
            st = st_of(j)
            lo = jnp.maximum(a, a0 + j * _CH) - st
            hi = jnp.minimum(aend, st + _CH) - st
            for dc in range(_D // 128):
                base = dc * 128

                def body(t, a8, base=base):
                    return tuple(
                        a8[q] + buf[t, pl.ds(base + q * _LANES, _LANES)]
                        for q in range(8)
                    )

                a8 = plsc.parallel_loop(lo, hi, 1, unroll=2, carry=(zero,) * 8)(body)
                for q in range(8):
                    plsc.addupdate(
                        acc.at[0, 0, pl.ds(base + q * _LANES, _LANES)], a8[q]
                    )

        start_dma(0, buf0, sem0)

        def pair_body(p, carry):
            j0 = 2 * p
            j1 = j0 + 1

            @pl.when(j1 < nch)
            def _():
                start_dma(j1, buf1, sem1)

            wait_dma(buf0, sem0)
            accum(buf0, j0)

            @pl.when(j1 < nch)
            def _():
                @pl.when(j1 + 1 < nch)
                def _():
                    start_dma(j1 + 1, buf0, sem0)

                wait_dma(buf1, sem1)
                accum(buf1, j1)

            return carry

        lax.fori_loop(0, (nch + 1) // 2, pair_body, 0)

        pltpu.sync_copy(acc, shparts.at[pl.ds(s, 1), pl.ds(i, 1)])
        return pe

    lax.while_loop(lambda r: r < r1, piece_body, r0)

    plsc.subcore_barrier()

    mine = jnp.where(c == 0, (s >= _SPLIT) & (s < k), s >= k)

    @pl.when(mine)
    def _():
        pltpu.sync_copy(shparts.at[pl.ds(0, _NT), pl.ds(s, 1)], tmp)
        lsplat = plsc.load_gather(len_v, [jnp.full((_LANES,), s, jnp.int32)])
        recip = jnp.full((_LANES,), 1.0, jnp.float32) / lsplat.astype(jnp.float32)
        for dc in range(_D // 128):
            base = dc * 128

            def body(t, a8, base=base):
                return tuple(
                    a8[q] + tmp[t, 0, pl.ds(base + q * _LANES, _LANES)]
                    for q in range(8)
                )

            a8 = plsc.parallel_loop(0, _NT, 1, unroll=2, carry=(zero,) * 8)(body)
            for q in range(8):
                orow[pl.ds(base + q * _LANES, _LANES)] = a8[q] * recip
        pltpu.sync_copy(orow, out.at[s])


def _tc_pool(lens_smem, feat_hbm, out_ref, bufs, sems):
    def nch(b):
        return (lens_smem[b] + _CHT - 1) // _CHT

    n_total = lax.fori_loop(0, _SPLIT, lambda b, t: t + nch(b), 0)

    def adv(b, j):
        last = j + 1 >= nch(b)
        return jnp.where(last, b + 1, b), jnp.where(last, 0, j + 1)

    def issue(b, j, q):
        pltpu.make_async_copy(
            feat_hbm.at[b, pl.ds(j * _CHT, _CHT), :], bufs.at[q], sems.at[q]
        ).start()

    def prologue(q, st):
        b, j = st

        @pl.when(q < n_total)
        def _():
            issue(b, j, q)

        return adv(b, j)

    bi0, ji0 = lax.fori_loop(0, _K, prologue, (0, 0))

    def body(t, st):
        bc, jc, bi, ji, acc = st
        q = t % _K
        pltpu.make_async_copy(
            feat_hbm.at[0, pl.ds(0, _CHT), :], bufs.at[q], sems.at[q]
        ).wait()
        blen = lens_smem[bc]
        m = (
            lax.broadcasted_iota(jnp.int32, (8, _CHT), 1) < (blen - jc * _CHT)
        ).astype(jnp.float32)
        acc = acc + jnp.dot(m, bufs[q], preferred_element_type=jnp.float32)
        last = jc + 1 >= nch(bc)

        @pl.when(last)
        def _():
            out_ref[pl.ds(bc, 1), :] = acc[0:1, :] / blen.astype(jnp.float32)

        acc = jnp.where(last, jnp.zeros_like(acc), acc)

        @pl.when(t + _K < n_total)
        def _():
            issue(bi, ji, q)

        bi, ji = adv(bi, ji)
        bc, jc = adv(bc, jc)
        return bc, jc, bi, ji, acc

    lax.fori_loop(
        0,
        n_total,
        body,
        (0, 0, bi0, ji0, jnp.zeros((8, _D), jnp.float32)),
    )


_tc_call = pl.pallas_call(
    _tc_pool,
    in_specs=[
        pl.BlockSpec(memory_space=pltpu.SMEM),
        pl.BlockSpec(memory_space=pl.ANY),
    ],
    out_specs=pl.BlockSpec(memory_space=pltpu.VMEM),
    out_shape=jax.ShapeDtypeStruct((_SPLIT, _D), jnp.float32),
    scratch_shapes=[
        pltpu.VMEM((_K, _CHT, _D), jnp.float32),
        pltpu.SemaphoreType.DMA((_K,)),
    ],
)


def kernel(features, lengths):
    tc_out = _tc_call(lengths, features)
    sc_out = _pool(features, lengths)
    return lax.dynamic_update_slice(sc_out, tc_out, (0, 0))
